# Initial kernel scaffold; baseline (speedup 1.0000x reference)
import jax
import jax.numpy as jnp
from jax.experimental import pallas as pl


def kernel(z, weight):
    raise NotImplementedError("write your pallas kernel here")


# speedup vs baseline: 1.2367x; 1.2367x over previous
import functools

import jax
import jax.numpy as jnp
from jax import lax
from jax.experimental import pallas as pl
from jax.experimental.pallas import tpu as pltpu
from jax.experimental.pallas import tpu_sc as plsc

NCODES = 8192
CDIM = 64
TOK = 8192
TM = 512
CB = 1024
NW = 32
TPW = TOK // NW


def _argmin_body(z_ref, w_ref, idx_ref):
    z = z_ref[...]
    zsq = jnp.sum(z * z, axis=1, keepdims=True)
    ids = lax.broadcasted_iota(jnp.int32, (TM, CB), 1).astype(jnp.float32)

    def step(c, carry):
        bval, bidx = carry
        wc = w_ref[pl.ds(c * CB, CB), :]
        wsq = jnp.sum(wc * wc, axis=1)
        p = lax.dot_general(z, wc, (((1,), (1,)), ((), ())),
                            preferred_element_type=jnp.float32)
        dist = (zsq + wsq[None, :]) - 2.0 * p
        lmin = jnp.min(dist, axis=1, keepdims=True)
        lidx = jnp.min(jnp.where(dist == lmin, ids, float(2 * CB)),
                       axis=1, keepdims=True)
        lidx = lidx + (c * CB).astype(jnp.float32)
        upd = lmin < bval
        return (jnp.where(upd, lmin, bval), jnp.where(upd, lidx, bidx))

    init = (jnp.full((TM, 1), jnp.inf, jnp.float32),
            jnp.zeros((TM, 1), jnp.float32))
    _, bidx = lax.fori_loop(0, NCODES // CB, step, init)
    idx_ref[...] = bidx.astype(jnp.int32)


_argmin_call = pl.pallas_call(
    _argmin_body,
    grid=(TOK // TM,),
    in_specs=[
        pl.BlockSpec((TM, CDIM), lambda i: (i, 0)),
        pl.BlockSpec((NCODES, CDIM), lambda i: (0, 0)),
    ],
    out_specs=pl.BlockSpec((TM, 1), lambda i: (i, 0)),
    out_shape=jax.ShapeDtypeStruct((TOK, 1), jnp.int32),
)


@functools.cache
def _sc_gather_hist_call():
    mesh = plsc.VectorSubcoreMesh(core_axis_name="c", subcore_axis_name="s")
    return pl.kernel(
        _sc_gather_hist,
        mesh=mesh,
        out_type=(
            jax.ShapeDtypeStruct((TOK, 128), jnp.float32),
            jax.ShapeDtypeStruct((2, NCODES), jnp.float32),
        ),
        scratch_types=(
            pltpu.VMEM((2, 128), jnp.int32),
            pltpu.VMEM((TPW, 128), jnp.float32),
            pltpu.VMEM((128,), jnp.float32),
            pltpu.VMEM_SHARED((NCODES,), jnp.float32),
            pltpu.SemaphoreType.DMA,
        ),
    )


def _sc_gather_hist(idx_hbm, w_hbm, zeros_hbm, zq_hbm, cnt_hbm,
                    idx_v, rows_v, ones_v, cnt_sh, sem):
    c = lax.axis_index("c")
    s = lax.axis_index("s")
    wid = s * 2 + c
    base = wid * TPW

    for t in range(8):
        ones_v[pl.ds(t * 16, 16)] = jnp.ones((16,), jnp.float32)

    pltpu.sync_copy(idx_hbm.at[wid], idx_v)

    cp0 = pltpu.async_copy(w_hbm.at[idx_v.at[0]], rows_v.at[pl.ds(0, 128)],
                           sem)
    cp1 = pltpu.async_copy(w_hbm.at[idx_v.at[1]], rows_v.at[pl.ds(128, 128)],
                           sem)

    @pl.when(s == 0)
    def _init_counts():
        pltpu.sync_copy(zeros_hbm, cnt_sh)

    cp0.wait()
    cp1.wait()
    pltpu.sync_copy(rows_v, zq_hbm.at[pl.ds(base, TPW)])

    plsc.subcore_barrier()
    pltpu.sync_copy(ones_v, cnt_sh.at[idx_v.at[0]], add=True)
    pltpu.sync_copy(ones_v, cnt_sh.at[idx_v.at[1]], add=True)
    plsc.subcore_barrier()

    @pl.when(s == 0)
    def _write_counts():
        pltpu.sync_copy(cnt_sh, cnt_hbm.at[c])


def _scalars_body(z_ref, qpad_ref, c_ref, loss_ref, perp_ref, zq_ref):
    q = qpad_ref[:, :CDIM]
    zq_ref[...] = q
    d = q - z_ref[...]
    loss = 0.25 * (jnp.sum(d * d) / float(TOK * CDIM))
    loss_ref[...] = loss.reshape(1, 1)
    avg = jnp.sum(c_ref[...], axis=0, keepdims=True) / float(TOK)
    ent = -jnp.sum(avg * jnp.log(avg + 1e-10))
    perp_ref[...] = jnp.exp(ent).reshape(1, 1)


_scalars_call = pl.pallas_call(
    _scalars_body,
    out_shape=(
        jax.ShapeDtypeStruct((1, 1), jnp.float32),
        jax.ShapeDtypeStruct((1, 1), jnp.float32),
        jax.ShapeDtypeStruct((TOK, CDIM), jnp.float32),
    ),
)


def kernel(z, weight):
    B, C, H, W = z.shape
    z_t = jnp.transpose(z, (0, 2, 3, 1))
    z_flat = z_t.reshape(-1, C)

    idx2d = _argmin_call(z_flat, weight)

    idx_r = idx2d.reshape(NW, 2, 128)
    zeros = jnp.zeros((NCODES,), jnp.float32)
    wpad = jnp.pad(weight, ((0, 0), (0, 128 - CDIM)))
    zq_pad, cnt = _sc_gather_hist_call()(idx_r, wpad, zeros)

    loss2, perp2, zq_flat = _scalars_call(z_flat, zq_pad, cnt)

    z_q = zq_flat.reshape(B, H, W, C).transpose(0, 3, 1, 2)
    return (z_q, loss2[0, 0], idx2d.reshape(B, H, W), perp2[0, 0])
